# baseline (speedup 1.0000x reference)
import math

import jax
import jax.numpy as jnp
from jax import lax
from jax.experimental import pallas as pl
from jax.experimental.pallas import tpu as pltpu

PI = float(math.pi)
THIRD = 2.0 * PI / 3.0
EPB = PI
DEV = 0.01
SPARSITY = 0.75
NH = 8
EPS = 1e-5

_F32 = jnp.float32

_CTH = [-0.8539859765994634, -0.5202960232130908, 0.42699298829972,
        0.08671600386018272, -0.035582749024575325, -0.004335799985469376,
        0.0011860916178656933, 0.00010323131987107525,
        -2.1179992460818492e-05, -1.4252774883987092e-06,
        2.3426863475342862e-07]
_SPR = [3.1415926535896856, -5.167712780003498, 2.5501640367064007,
        -0.5992644488554889, 0.08214491942222915, -0.007364482642017926,
        0.00044817209749427485]
_CR = [0.9999999999973345, -0.4999999999757866, 0.04166666661291515,
       -0.001388888838246469, 2.480156236069027e-05,
       -2.7556612807725635e-07, 2.0864819614516772e-09,
       -1.1351627719367773e-11, 4.127357214685606e-14]


def _horner(coefs, x):
    acc = jnp.full_like(x, jnp.float32(coefs[-1]))
    for c in coefs[-2::-1]:
        acc = acc * x + jnp.float32(c)
    return acc


def _prune_body(w_ref, o_ref, a_ref):
    R = w_ref.shape[1]
    s = R * 128
    w = w_ref[0]
    row = lax.broadcasted_iota(jnp.int32, (R, 128), 0)
    col = lax.broadcasted_iota(jnp.int32, (R, 128), 1)
    f = (row * 128 + col).astype(_F32)
    n = jnp.round(w)
    r = w - n
    odd = (n.astype(jnp.int32) & 1) != 0
    r = jnp.where(odd, -r, r)
    sinpw = r * _horner(_SPR, r * r)
    bloom = jnp.clip(sinpw, -1.0, 1.0)
    t = f * jnp.float32(1.0 / (s - 1)) - 0.5
    bloom = bloom + bloom * _horner(_CTH, t) * 1.5
    u = bloom * (EPB * EPB)
    mf = jnp.round(u * jnp.float32(1.0 / (2.0 * PI)))
    rr = (u - mf * jnp.float32(6.28125)) - mf * jnp.float32(
        2.0 * PI - 6.28125)
    etched = _horner(_CR, rr * rr) + bloom * bloom * (DEV / PI)
    a_ref[...] = jnp.abs(etched)

    pos = (s - 1) * (1.0 - SPARSITY)
    k = int(math.floor(pos))
    frac = jnp.float32(pos - k)
    kp1 = jnp.float32(k + 1)
    kp2 = jnp.float32(k + 2)

    def count_le(t):
        bits = lax.bitcast_convert_type(a_ref[...], jnp.int32)
        return jnp.sum((bits <= t).astype(_F32))

    def bs_body(_, carry):
        lo, hi = carry
        mid = lo + ((hi - lo) >> 1)
        pred = count_le(mid) >= kp1
        hi = jnp.where(pred, mid, hi)
        lo = jnp.where(pred, lo, mid + 1)
        return lo, hi

    lo0 = jnp.int32(0)
    hi0 = jnp.int32(0x3F840000)
    lo, hi = lax.fori_loop(0, 30, bs_body, (lo0, hi0))
    vk = hi

    a = a_ref[...]
    bits = lax.bitcast_convert_type(a, jnp.int32)
    le = bits <= vk
    c = jnp.sum(le.astype(_F32))
    a_k = jnp.max(jnp.where(le, a, jnp.float32(-1.0)))
    a_k1_gt = jnp.min(jnp.where(le, jnp.float32(3.0e38), a))
    a_k1 = jnp.where(c >= kp2, a_k, a_k1_gt)
    thr = a_k + (a_k1 - a_k) * frac
    o_ref[0] = w * (a > thr).astype(_F32)


def _prune_flat(wf):
    G, R, _ = wf.shape
    return pl.pallas_call(
        _prune_body,
        out_shape=jax.ShapeDtypeStruct((G, R, 128), _F32),
        grid=(G,),
        in_specs=[pl.BlockSpec((1, R, 128), lambda l: (l, 0, 0))],
        out_specs=pl.BlockSpec((1, R, 128), lambda l: (l, 0, 0)),
        scratch_shapes=[pltpu.VMEM((R, 128), _F32)],
        compiler_params=pltpu.CompilerParams(
            dimension_semantics=("arbitrary",),
            vmem_limit_bytes=48 * 1024 * 1024,
        ),
        name="prune",
    )(wf)


def _prune(wl):
    L, M, N = wl.shape
    R = (M * N) // 128
    return _prune_flat(wl.reshape(L, R, 128)).reshape(L, M, N)


def _embed_body(src_ref, emb_ref, pos_ref, o_ref):
    b = pl.program_id(0)
    S = o_ref.shape[0]
    D = o_ref.shape[2]
    scale = jnp.sqrt(jnp.float32(D))
    for i in range(S):
        idx = src_ref[b, i]
        o_ref[i, 0, :] = emb_ref[idx, 0, :] * scale + pos_ref[i, 0, :]


def _embed(src, emb, pos):
    B, S = src.shape
    V, D = emb.shape
    emb3 = emb.reshape(V, 1, D)
    pos3 = pos[0, :S, :].reshape(S, 1, D)
    out = pl.pallas_call(
        _embed_body,
        out_shape=jax.ShapeDtypeStruct((B * S, 1, D), _F32),
        grid=(B,),
        in_specs=[
            pl.BlockSpec(memory_space=pltpu.SMEM),
            pl.BlockSpec((V, 1, D), lambda b: (0, 0, 0)),
            pl.BlockSpec((S, 1, D), lambda b: (0, 0, 0)),
        ],
        out_specs=pl.BlockSpec((S, 1, D), lambda b: (b, 0, 0)),
        compiler_params=pltpu.CompilerParams(
            dimension_semantics=("arbitrary",),
            vmem_limit_bytes=56 * 1024 * 1024,
        ),
        name="embed",
    )(src, emb3, pos3)
    return out.reshape(B, S, D)


def _layernorm(t, g, b):
    m = jnp.mean(t, axis=-1, keepdims=True)
    v = jnp.mean((t - m) ** 2, axis=-1, keepdims=True)
    return (t - m) * lax.rsqrt(v + EPS) * g + b


def _layer_body(x_ref, wiT_ref, bi_ref, woT_ref, bo_ref, g1_ref, b1_ref,
                g2_ref, b2_ref, w1T_ref, bf1_ref, w2T_ref, bf2_ref,
                o_ref, qkv_s, att_s):
    S, D = x_ref.shape[1], x_ref.shape[2]
    dh = D // NH
    bf16 = jnp.bfloat16
    x = x_ref[0]
    qkv_s[...] = (
        jnp.dot(x.astype(bf16), wiT_ref[...], preferred_element_type=_F32)
        + bi_ref[...]
    )
    inv_sqrt_dh = jnp.float32(1.0) / jnp.sqrt(jnp.float32(dh))
    for h in range(NH):
        q = qkv_s[:, h * dh:(h + 1) * dh].astype(bf16)
        k = qkv_s[:, D + h * dh:D + (h + 1) * dh].astype(bf16)
        v = qkv_s[:, 2 * D + h * dh:2 * D + (h + 1) * dh].astype(bf16)
        sc = lax.dot_general(q, k, (((1,), (1,)), ((), ())),
                             preferred_element_type=_F32) * inv_sqrt_dh
        m = jnp.max(sc, axis=-1, keepdims=True)
        e = jnp.exp(sc - m)
        p = e / jnp.sum(e, axis=-1, keepdims=True)
        att_s[:, h * dh:(h + 1) * dh] = jnp.dot(
            p.astype(bf16), v, preferred_element_type=_F32)
    o = jnp.dot(att_s[...].astype(bf16), woT_ref[...],
                preferred_element_type=_F32)
    o = o + bo_ref[...]
    x1 = _layernorm(x + o, g1_ref[...], b1_ref[...])
    h1 = jax.nn.relu(
        jnp.dot(x1.astype(bf16), w1T_ref[...], preferred_element_type=_F32)
        + bf1_ref[...])
    ff = jnp.dot(h1.astype(bf16), w2T_ref[...], preferred_element_type=_F32)
    ff = ff + bf2_ref[...]
    x2 = _layernorm(x1 + ff, g2_ref[...], b2_ref[...])
    o_ref[0] = x2


def _layer(x, wiT, bi, woT, bo, g1, b1, g2, b2, w1T, bf1, w2T, bf2):
    B, S, D = x.shape
    F = w1T.shape[1]
    return pl.pallas_call(
        _layer_body,
        out_shape=jax.ShapeDtypeStruct((B, S, D), _F32),
        grid=(B,),
        in_specs=[
            pl.BlockSpec((1, S, D), lambda b: (b, 0, 0)),
            pl.BlockSpec((D, 3 * D), lambda b: (0, 0)),
            pl.BlockSpec((1, 3 * D), lambda b: (0, 0)),
            pl.BlockSpec((D, D), lambda b: (0, 0)),
            pl.BlockSpec((1, D), lambda b: (0, 0)),
            pl.BlockSpec((1, D), lambda b: (0, 0)),
            pl.BlockSpec((1, D), lambda b: (0, 0)),
            pl.BlockSpec((1, D), lambda b: (0, 0)),
            pl.BlockSpec((1, D), lambda b: (0, 0)),
            pl.BlockSpec((D, F), lambda b: (0, 0)),
            pl.BlockSpec((1, F), lambda b: (0, 0)),
            pl.BlockSpec((F, D), lambda b: (0, 0)),
            pl.BlockSpec((1, D), lambda b: (0, 0)),
        ],
        out_specs=pl.BlockSpec((1, S, D), lambda b: (b, 0, 0)),
        scratch_shapes=[
            pltpu.VMEM((S, 3 * D), _F32),
            pltpu.VMEM((S, D), _F32),
        ],
        compiler_params=pltpu.CompilerParams(
            dimension_semantics=("arbitrary",),
            vmem_limit_bytes=56 * 1024 * 1024,
        ),
        name="layer",
    )(x, wiT, bi, woT, bo, g1, b1, g2, b2, w1T, bf1, w2T, bf2)


def _make_logits_body(v_tile, V, J):
    tail = V - (J - 1) * v_tile

    def _logits_body(x_ref, wT_ref, b_ref, o_ref):
        j = pl.program_id(1)
        val = (
            jnp.dot(x_ref[0].astype(jnp.bfloat16), wT_ref[...],
                    preferred_element_type=_F32)
            + b_ref[...]
        )

        if J > 1:
            @pl.when(j < J - 1)
            def _():
                off = pl.multiple_of(j * v_tile, 128)
                o_ref[0, :, pl.ds(off, v_tile)] = val

            @pl.when(j == J - 1)
            def _():
                o_ref[0, :, pl.ds((J - 1) * v_tile, tail)] = val[:, :tail]
        else:
            o_ref[0, :, :] = val[:, :tail]

    return _logits_body


def _logits(x, out_w, out_b, v_tile):
    B, S, D = x.shape
    V = out_w.shape[0]
    Vp = ((V + v_tile - 1) // v_tile) * v_tile
    wT = jnp.pad(out_w, ((0, Vp - V), (0, 0))).T.astype(jnp.bfloat16)
    bp = jnp.pad(out_b, (0, Vp - V)).reshape(1, Vp)
    J = Vp // v_tile
    return pl.pallas_call(
        _make_logits_body(v_tile, V, J),
        out_shape=jax.ShapeDtypeStruct((B, S, V), _F32),
        grid=(B, J),
        in_specs=[
            pl.BlockSpec((1, S, D), lambda b, j: (b, 0, 0)),
            pl.BlockSpec((D, v_tile), lambda b, j: (0, j)),
            pl.BlockSpec((1, v_tile), lambda b, j: (0, j)),
        ],
        out_specs=pl.BlockSpec((1, S, V), lambda b, j: (b, 0, 0)),
        compiler_params=pltpu.CompilerParams(
            dimension_semantics=("arbitrary", "arbitrary"),
            vmem_limit_bytes=56 * 1024 * 1024,
        ),
        name="logits",
    )(x, wT, bp)


def kernel(src, emb, pos, in_proj_w, in_proj_b, out_proj_w, out_proj_b,
           ln1_g, ln1_b, ln2_g, ln2_b, lin1_w, lin1_b, lin2_w, lin2_b,
           out_w, out_b):
    B, S = src.shape
    D = emb.shape[1]
    L = in_proj_w.shape[0]

    wi_m = _prune(in_proj_w)
    F = lin1_w.shape[1]
    R12 = (F * D) // 128
    w12 = jnp.concatenate(
        [lin1_w.reshape(L, R12, 128), lin2_w.reshape(L, R12, 128)], axis=0)
    w12_m = _prune_flat(w12)
    w1_m = w12_m[:L].reshape(L, F, D)
    w2_m = w12_m[L:].reshape(L, D, F)

    bf16 = jnp.bfloat16
    wiT = jnp.swapaxes(wi_m, 1, 2).astype(bf16)
    woT = jnp.swapaxes(out_proj_w, 1, 2).astype(bf16)
    w1T = jnp.swapaxes(w1_m, 1, 2).astype(bf16)
    w2T = jnp.swapaxes(w2_m, 1, 2).astype(bf16)

    x = _embed(src, emb, pos)

    for l in range(L):
        x = _layer(
            x,
            wiT[l], in_proj_b[l].reshape(1, -1),
            woT[l], out_proj_b[l].reshape(1, -1),
            ln1_g[l].reshape(1, -1), ln1_b[l].reshape(1, -1),
            ln2_g[l].reshape(1, -1), ln2_b[l].reshape(1, -1),
            w1T[l], lin1_b[l].reshape(1, -1),
            w2T[l], lin2_b[l].reshape(1, -1),
        )

    return _logits(x, out_w, out_b, v_tile=1280)


# speedup vs baseline: 1.0333x; 1.0333x over previous
import math

import jax
import jax.numpy as jnp
from jax import lax
from jax.experimental import pallas as pl
from jax.experimental.pallas import tpu as pltpu

PI = float(math.pi)
THIRD = 2.0 * PI / 3.0
EPB = PI
DEV = 0.01
SPARSITY = 0.75
NH = 8
EPS = 1e-5

_F32 = jnp.float32

_CTH = [-0.8539859765994634, -0.5202960232130908, 0.42699298829972,
        0.08671600386018272, -0.035582749024575325, -0.004335799985469376,
        0.0011860916178656933, 0.00010323131987107525,
        -2.1179992460818492e-05, -1.4252774883987092e-06,
        2.3426863475342862e-07]
_SPR = [3.1415926535896856, -5.167712780003498, 2.5501640367064007,
        -0.5992644488554889, 0.08214491942222915, -0.007364482642017926,
        0.00044817209749427485]
_CR = [0.9999999999973345, -0.4999999999757866, 0.04166666661291515,
       -0.001388888838246469, 2.480156236069027e-05,
       -2.7556612807725635e-07, 2.0864819614516772e-09,
       -1.1351627719367773e-11, 4.127357214685606e-14]


def _horner(coefs, x):
    acc = jnp.full_like(x, jnp.float32(coefs[-1]))
    for c in coefs[-2::-1]:
        acc = acc * x + jnp.float32(c)
    return acc


def _prune_body(w_ref, o_ref, a_ref):
    R = w_ref.shape[1]
    s = R * 128
    w = w_ref[0]
    row = lax.broadcasted_iota(jnp.int32, (R, 128), 0)
    col = lax.broadcasted_iota(jnp.int32, (R, 128), 1)
    f = (row * 128 + col).astype(_F32)
    n = jnp.round(w)
    r = w - n
    odd = (n.astype(jnp.int32) & 1) != 0
    r = jnp.where(odd, -r, r)
    sinpw = r * _horner(_SPR, r * r)
    bloom = jnp.clip(sinpw, -1.0, 1.0)
    t = f * jnp.float32(1.0 / (s - 1)) - 0.5
    bloom = bloom + bloom * _horner(_CTH, t) * 1.5
    u = bloom * (EPB * EPB)
    mf = jnp.round(u * jnp.float32(1.0 / (2.0 * PI)))
    rr = (u - mf * jnp.float32(6.28125)) - mf * jnp.float32(
        2.0 * PI - 6.28125)
    etched = _horner(_CR, rr * rr) + bloom * bloom * (DEV / PI)
    a_ref[...] = jnp.abs(etched)

    pos = (s - 1) * (1.0 - SPARSITY)
    k = int(math.floor(pos))
    frac = jnp.float32(pos - k)
    kp1 = jnp.float32(k + 1)
    kp2 = jnp.float32(k + 2)

    def count_le(t):
        bits = lax.bitcast_convert_type(a_ref[...], jnp.int32)
        return jnp.sum((bits <= t).astype(_F32))

    def bs_body(_, carry):
        lo, hi = carry
        mid = lo + ((hi - lo) >> 1)
        pred = count_le(mid) >= kp1
        hi = jnp.where(pred, mid, hi)
        lo = jnp.where(pred, lo, mid + 1)
        return lo, hi

    lo0 = jnp.int32(0)
    hi0 = jnp.int32(0x3F840000)
    lo, hi = lax.fori_loop(0, 30, bs_body, (lo0, hi0))
    vk = hi

    a = a_ref[...]
    bits = lax.bitcast_convert_type(a, jnp.int32)
    le = bits <= vk
    c = jnp.sum(le.astype(_F32))
    a_k = jnp.max(jnp.where(le, a, jnp.float32(-1.0)))
    a_k1_gt = jnp.min(jnp.where(le, jnp.float32(3.0e38), a))
    a_k1 = jnp.where(c >= kp2, a_k, a_k1_gt)
    thr = a_k + (a_k1 - a_k) * frac
    o_ref[0] = w * (a > thr).astype(_F32)


def _prune_flat(wf):
    G, R, _ = wf.shape
    return pl.pallas_call(
        _prune_body,
        out_shape=jax.ShapeDtypeStruct((G, R, 128), _F32),
        grid=(G,),
        in_specs=[pl.BlockSpec((1, R, 128), lambda l: (l, 0, 0))],
        out_specs=pl.BlockSpec((1, R, 128), lambda l: (l, 0, 0)),
        scratch_shapes=[pltpu.VMEM((R, 128), _F32)],
        compiler_params=pltpu.CompilerParams(
            dimension_semantics=("arbitrary",),
            vmem_limit_bytes=48 * 1024 * 1024,
        ),
        name="prune",
    )(wf)


def _prune(wl):
    L, M, N = wl.shape
    R = (M * N) // 128
    return _prune_flat(wl.reshape(L, R, 128)).reshape(L, M, N)


def _embed_body(src_ref, emb_ref, pos_ref, o_ref):
    b = pl.program_id(0)
    S = o_ref.shape[0]
    D = o_ref.shape[2]
    scale = jnp.sqrt(jnp.float32(D))
    for i in range(S):
        idx = src_ref[b, i]
        o_ref[i, 0, :] = emb_ref[idx, 0, :] * scale + pos_ref[i, 0, :]


def _embed(src, emb, pos):
    B, S = src.shape
    V, D = emb.shape
    emb3 = emb.reshape(V, 1, D)
    pos3 = pos[0, :S, :].reshape(S, 1, D)
    out = pl.pallas_call(
        _embed_body,
        out_shape=jax.ShapeDtypeStruct((B * S, 1, D), _F32),
        grid=(B,),
        in_specs=[
            pl.BlockSpec(memory_space=pltpu.SMEM),
            pl.BlockSpec((V, 1, D), lambda b: (0, 0, 0)),
            pl.BlockSpec((S, 1, D), lambda b: (0, 0, 0)),
        ],
        out_specs=pl.BlockSpec((S, 1, D), lambda b: (b, 0, 0)),
        compiler_params=pltpu.CompilerParams(
            dimension_semantics=("arbitrary",),
            vmem_limit_bytes=56 * 1024 * 1024,
        ),
        name="embed",
    )(src, emb3, pos3)
    return out.reshape(B, S, D)


def _layernorm(t, g, b):
    m = jnp.mean(t, axis=-1, keepdims=True)
    v = jnp.mean((t - m) ** 2, axis=-1, keepdims=True)
    return (t - m) * lax.rsqrt(v + EPS) * g + b


def _layer_body(x_ref, wiT_ref, bi_ref, woT_ref, bo_ref, g1_ref, b1_ref,
                g2_ref, b2_ref, w1T_ref, bf1_ref, w2T_ref, bf2_ref,
                o_ref, qkv_s, att_s):
    S, D = x_ref.shape[1], x_ref.shape[2]
    dh = D // NH
    bf16 = jnp.bfloat16
    x = x_ref[0]
    qkv_s[...] = (
        jnp.dot(x.astype(bf16), wiT_ref[...], preferred_element_type=_F32)
        + bi_ref[...]
    )
    inv_sqrt_dh = jnp.float32(1.0) / jnp.sqrt(jnp.float32(dh))
    for h in range(NH):
        q = qkv_s[:, h * dh:(h + 1) * dh].astype(bf16)
        k = qkv_s[:, D + h * dh:D + (h + 1) * dh].astype(bf16)
        v = qkv_s[:, 2 * D + h * dh:2 * D + (h + 1) * dh].astype(bf16)
        sc = lax.dot_general(q, k, (((1,), (1,)), ((), ())),
                             preferred_element_type=_F32) * inv_sqrt_dh
        m = jnp.max(sc, axis=-1, keepdims=True)
        e = jnp.exp(sc - m)
        p = e / jnp.sum(e, axis=-1, keepdims=True)
        att_s[:, h * dh:(h + 1) * dh] = jnp.dot(
            p.astype(bf16), v, preferred_element_type=_F32)
    o = jnp.dot(att_s[...].astype(bf16), woT_ref[...],
                preferred_element_type=_F32)
    o = o + bo_ref[...]
    x1 = _layernorm(x + o, g1_ref[...], b1_ref[...])
    h1 = jax.nn.relu(
        jnp.dot(x1.astype(bf16), w1T_ref[...], preferred_element_type=_F32)
        + bf1_ref[...])
    ff = jnp.dot(h1.astype(bf16), w2T_ref[...], preferred_element_type=_F32)
    ff = ff + bf2_ref[...]
    x2 = _layernorm(x1 + ff, g2_ref[...], b2_ref[...])
    o_ref[0] = x2


def _layer(x, wiT, bi, woT, bo, g1, b1, g2, b2, w1T, bf1, w2T, bf2):
    B, S, D = x.shape
    F = w1T.shape[1]
    return pl.pallas_call(
        _layer_body,
        out_shape=jax.ShapeDtypeStruct((B, S, D), _F32),
        grid=(B,),
        in_specs=[
            pl.BlockSpec((1, S, D), lambda b: (b, 0, 0)),
            pl.BlockSpec((D, 3 * D), lambda b: (0, 0)),
            pl.BlockSpec((1, 3 * D), lambda b: (0, 0)),
            pl.BlockSpec((D, D), lambda b: (0, 0)),
            pl.BlockSpec((1, D), lambda b: (0, 0)),
            pl.BlockSpec((1, D), lambda b: (0, 0)),
            pl.BlockSpec((1, D), lambda b: (0, 0)),
            pl.BlockSpec((1, D), lambda b: (0, 0)),
            pl.BlockSpec((1, D), lambda b: (0, 0)),
            pl.BlockSpec((D, F), lambda b: (0, 0)),
            pl.BlockSpec((1, F), lambda b: (0, 0)),
            pl.BlockSpec((F, D), lambda b: (0, 0)),
            pl.BlockSpec((1, D), lambda b: (0, 0)),
        ],
        out_specs=pl.BlockSpec((1, S, D), lambda b: (b, 0, 0)),
        scratch_shapes=[
            pltpu.VMEM((S, 3 * D), _F32),
            pltpu.VMEM((S, D), _F32),
        ],
        compiler_params=pltpu.CompilerParams(
            dimension_semantics=("arbitrary",),
            vmem_limit_bytes=56 * 1024 * 1024,
        ),
        name="layer",
    )(x, wiT, bi, woT, bo, g1, b1, g2, b2, w1T, bf1, w2T, bf2)


def _logits_body(x_ref, wT_ref, b_ref, o_ref):
    o_ref[0] = (
        jnp.dot(x_ref[0].astype(jnp.bfloat16), wT_ref[...],
                preferred_element_type=_F32)
        + b_ref[...]
    )


def _logits(x, out_w, out_b, v_tile):
    B, S, D = x.shape
    V = out_w.shape[0]
    Vp = ((V + v_tile - 1) // v_tile) * v_tile
    wT = jnp.pad(out_w, ((0, Vp - V), (0, 0))).T.astype(jnp.bfloat16)
    bp = jnp.pad(out_b, (0, Vp - V)).reshape(1, Vp)
    J = Vp // v_tile
    return pl.pallas_call(
        _logits_body,
        out_shape=jax.ShapeDtypeStruct((B, S, V), _F32),
        grid=(B, J),
        in_specs=[
            pl.BlockSpec((1, S, D), lambda b, j: (b, 0, 0)),
            pl.BlockSpec((D, v_tile), lambda b, j: (0, j)),
            pl.BlockSpec((1, v_tile), lambda b, j: (0, j)),
        ],
        out_specs=pl.BlockSpec((1, S, v_tile), lambda b, j: (b, 0, j)),
        compiler_params=pltpu.CompilerParams(
            dimension_semantics=("arbitrary", "arbitrary"),
            vmem_limit_bytes=56 * 1024 * 1024,
        ),
        name="logits",
    )(x, wT, bp)


def kernel(src, emb, pos, in_proj_w, in_proj_b, out_proj_w, out_proj_b,
           ln1_g, ln1_b, ln2_g, ln2_b, lin1_w, lin1_b, lin2_w, lin2_b,
           out_w, out_b):
    B, S = src.shape
    D = emb.shape[1]
    L = in_proj_w.shape[0]

    wi_m = _prune(in_proj_w)
    w1_m = _prune(lin1_w)
    w2_m = _prune(lin2_w)

    bf16 = jnp.bfloat16
    wiT = jnp.swapaxes(wi_m, 1, 2).astype(bf16)
    woT = jnp.swapaxes(out_proj_w, 1, 2).astype(bf16)
    w1T = jnp.swapaxes(w1_m, 1, 2).astype(bf16)
    w2T = jnp.swapaxes(w2_m, 1, 2).astype(bf16)

    x = _embed(src, emb, pos)

    for l in range(L):
        x = _layer(
            x,
            wiT[l], in_proj_b[l].reshape(1, -1),
            woT[l], out_proj_b[l].reshape(1, -1),
            ln1_g[l].reshape(1, -1), ln1_b[l].reshape(1, -1),
            ln2_g[l].reshape(1, -1), ln2_b[l].reshape(1, -1),
            w1T[l], lin1_b[l].reshape(1, -1),
            w2T[l], lin2_b[l].reshape(1, -1),
        )

    return _logits(x, out_w, out_b, v_tile=1280)
